# baseline (speedup 1.0000x reference)
import functools

import jax
import jax.numpy as jnp
from jax.experimental import pallas as pl
from jax.experimental.pallas import tpu as pltpu


def _mm(x, w):
    return jnp.dot(x.astype(jnp.bfloat16), w.astype(jnp.bfloat16),
                   preferred_element_type=jnp.float32)


def _mm3(a, b):
    ah = a.astype(jnp.bfloat16)
    al = (a - ah.astype(jnp.float32)).astype(jnp.bfloat16)
    bh = b.astype(jnp.bfloat16)
    bl = (b - bh.astype(jnp.float32)).astype(jnp.bfloat16)
    f32 = jnp.float32
    return (jnp.dot(ah, bh, preferred_element_type=f32)
            + (jnp.dot(al, bh, preferred_element_type=f32)
               + jnp.dot(ah, bl, preferred_element_type=f32)))


def _stage_kernel(nlayers, c1, acts, *refs):
    q_ref, kt_ref, p1_ref, p2_ref = refs[:4]
    wrefs = refs[4:4 + 2 * nlayers]
    out_ref = refs[-1]

    t = q_ref.shape[1]
    n2 = kt_ref.shape[2]

    q = q_ref[0, :, :]
    kt = kt_ref[0, :, :]
    aa = jnp.sum(q * q, axis=1, keepdims=True)
    bb = jnp.sum(kt * kt, axis=0, keepdims=True)
    ab = _mm(q, kt)
    d = jnp.maximum(aa + bb - 2.0 * ab, 0.0)

    iota = jax.lax.broadcasted_iota(jnp.int32, (t, n2), 1).astype(jnp.float32)
    big = jnp.float32(3.0e38)
    nf = jnp.float32(n2)
    mn1 = jnp.min(d, axis=1, keepdims=True)
    am1 = jnp.min(jnp.where(d == mn1, iota, nf), axis=1, keepdims=True)
    e1 = iota == am1
    mn2 = jnp.min(jnp.where(e1, big, d), axis=1, keepdims=True)
    am2 = jnp.min(jnp.where((d == mn2) & ~e1, iota, nf), axis=1, keepdims=True)
    e2 = iota == am2
    e12 = e1 | e2
    mn3 = jnp.min(jnp.where(e12, big, d), axis=1, keepdims=True)
    am3 = jnp.min(jnp.where((d == mn3) & ~e12, iota, nf), axis=1, keepdims=True)
    e3 = iota == am3
    ohs = [e1.astype(jnp.bfloat16), e2.astype(jnp.bfloat16),
           e3.astype(jnp.bfloat16)]
    wks = [1.0 / jnp.maximum(mn, 1e-10) for mn in (mn1, mn2, mn3)]
    wtot = wks[0] + wks[1] + wks[2]

    f32 = jnp.float32
    p2 = p2_ref[0, :, :]
    p2a = p2.astype(jnp.bfloat16)
    r1 = p2 - p2a.astype(f32)
    p2b = r1.astype(jnp.bfloat16)
    p2c = (r1 - p2b.astype(f32)).astype(jnp.bfloat16)
    interp = None
    for k in range(3):
        g = (jnp.dot(ohs[k], p2a, preferred_element_type=f32)
             + jnp.dot(ohs[k], p2b, preferred_element_type=f32)
             + jnp.dot(ohs[k], p2c, preferred_element_type=f32))
        term = g * (wks[k] / wtot)
        interp = term if interp is None else interp + term

    w0 = wrefs[0]
    b0 = wrefs[1]
    x = (_mm(p1_ref[0, :, :], w0[:c1, :]) + _mm(interp, w0[c1:, :])
         + b0[0:1, :])
    x = jnp.maximum(x, 0.0) if acts[0] == 'r' else jnp.tanh(x)
    for i in range(1, nlayers):
        w = wrefs[2 * i]
        b = wrefs[2 * i + 1]
        x = _mm(x, w[:, :]) + b[0:1, :]
        x = jnp.maximum(x, 0.0) if acts[i] == 'r' else jnp.tanh(x)
    out_ref[0, :, :] = x


def _fp_stage(xyz1, xyz2, points1, points2, layers, acts, tile):
    bb, n1, _ = xyz1.shape
    n2 = xyz2.shape[1]
    c1 = points1.shape[2]
    c2 = points2.shape[2]
    out_dim = layers[-1][0].shape[1]
    kt = jnp.swapaxes(xyz2, 1, 2)

    args = [xyz1, kt, points1, points2]
    in_specs = [
        pl.BlockSpec((1, tile, 3), lambda b, s: (b, s, 0)),
        pl.BlockSpec((1, 3, n2), lambda b, s: (b, 0, 0)),
        pl.BlockSpec((1, tile, c1), lambda b, s: (b, s, 0)),
        pl.BlockSpec((1, n2, c2), lambda b, s: (b, 0, 0)),
    ]
    for w, b in layers:
        args += [w, b.reshape(1, -1)]
        in_specs.append(pl.BlockSpec(w.shape, lambda b, s: (0, 0)))
        in_specs.append(pl.BlockSpec((1, w.shape[1]), lambda b, s: (0, 0)))

    return pl.pallas_call(
        functools.partial(_stage_kernel, len(layers), c1, acts),
        grid=(bb, n1 // tile),
        in_specs=in_specs,
        out_specs=pl.BlockSpec((1, tile, out_dim), lambda b, s: (b, s, 0)),
        out_shape=jax.ShapeDtypeStruct((bb, n1, out_dim), jnp.float32),
        compiler_params=pltpu.CompilerParams(
            dimension_semantics=("parallel", "arbitrary")),
    )(*args)


def kernel(xyz0, xyz1, xyz2, xyz3, xyz4, points0, points1, points2, points3,
           points4, fa1_W0, fa1_b0, fa1_W1, fa1_b1, fa2_W0, fa2_b0, fa2_W1,
           fa2_b1, fa3_W0, fa3_b0, fa3_W1, fa3_b1, fa4_W0, fa4_b0, fa4_W1,
           fa4_b1, fa4_W2, fa4_b2, head_W0, head_b0, head_W1, head_b1,
           head_W2, head_b2):
    p3 = _fp_stage(xyz3, xyz4, points3, points4,
                   [(fa1_W0, fa1_b0), (fa1_W1, fa1_b1)], 'rr', 64)
    p2 = _fp_stage(xyz2, xyz3, points2, p3,
                   [(fa2_W0, fa2_b0), (fa2_W1, fa2_b1)], 'rr', 256)
    p1 = _fp_stage(xyz1, xyz2, points1, p2,
                   [(fa3_W0, fa3_b0), (fa3_W1, fa3_b1)], 'rr', 1024)
    head_W2p = jnp.pad(head_W2, ((0, 0), (0, 5)))
    head_b2p = jnp.pad(head_b2, (0, 5))
    out8 = _fp_stage(xyz0, xyz1, points0, p1,
                     [(fa4_W0, fa4_b0), (fa4_W1, fa4_b1), (fa4_W2, fa4_b2),
                      (head_W0, head_b0), (head_W1, head_b1),
                      (head_W2p, head_b2p)], 'rrrrrt', 512)
    return out8[..., :3]


# speedup vs baseline: 1.3200x; 1.3200x over previous
import functools

import jax
import jax.numpy as jnp
from jax.experimental import pallas as pl
from jax.experimental.pallas import tpu as pltpu


def _mm(x, w):
    return jnp.dot(x.astype(jnp.bfloat16), w.astype(jnp.bfloat16),
                   preferred_element_type=jnp.float32)


def _mm3(a, b):
    ah = a.astype(jnp.bfloat16)
    al = (a - ah.astype(jnp.float32)).astype(jnp.bfloat16)
    bh = b.astype(jnp.bfloat16)
    bl = (b - bh.astype(jnp.float32)).astype(jnp.bfloat16)
    f32 = jnp.float32
    return (jnp.dot(ah, bh, preferred_element_type=f32)
            + (jnp.dot(al, bh, preferred_element_type=f32)
               + jnp.dot(ah, bl, preferred_element_type=f32)))


def _stage_kernel(nlayers, c1, acts, *refs):
    q_ref, kt_ref, p1_ref, p2_ref = refs[:4]
    wrefs = refs[4:4 + 2 * nlayers]
    out_ref = refs[-1]

    t = q_ref.shape[1]
    n2 = kt_ref.shape[2]

    q = q_ref[0, :, :]
    kt = kt_ref[0, :, :]
    aa = jnp.sum(q * q, axis=1, keepdims=True)
    bb = jnp.sum(kt * kt, axis=0, keepdims=True)
    ab = _mm(q, kt)
    d = jnp.maximum(aa + bb - 2.0 * ab, 0.0)

    iota = jax.lax.broadcasted_iota(jnp.int32, (t, n2), 1).astype(jnp.float32)
    big = jnp.float32(3.0e38)
    nf = jnp.float32(n2)
    dw = d
    ohs = []
    wks = []
    for k in range(3):
        mn = jnp.min(dw, axis=1, keepdims=True)
        am = jnp.min(jnp.where(dw == mn, iota, nf), axis=1, keepdims=True)
        oh = iota == am
        wks.append(1.0 / jnp.maximum(mn, 1e-10))
        ohs.append(oh.astype(jnp.bfloat16))
        if k < 2:
            dw = jnp.where(oh, big, dw)
    wtot = wks[0] + wks[1] + wks[2]

    f32 = jnp.float32
    p2 = p2_ref[0, :, :]
    p2a = p2.astype(jnp.bfloat16)
    r1 = p2 - p2a.astype(f32)
    p2b = r1.astype(jnp.bfloat16)
    p2c = (r1 - p2b.astype(f32)).astype(jnp.bfloat16)
    c2 = p2.shape[1]
    ohcat = jnp.concatenate(ohs, axis=0)
    p2cat = jnp.concatenate([p2a, p2b, p2c], axis=1)
    gcat = jnp.dot(ohcat, p2cat, preferred_element_type=f32)
    interp = None
    for k in range(3):
        gk = gcat[k * t:(k + 1) * t, :]
        g = gk[:, :c2] + gk[:, c2:2 * c2] + gk[:, 2 * c2:]
        term = g * (wks[k] / wtot)
        interp = term if interp is None else interp + term

    w0 = wrefs[0]
    b0 = wrefs[1]
    x = (_mm(p1_ref[0, :, :], w0[:c1, :]) + _mm(interp, w0[c1:, :])
         + b0[0:1, :])
    x = jnp.maximum(x, 0.0) if acts[0] == 'r' else jnp.tanh(x)
    for i in range(1, nlayers):
        w = wrefs[2 * i]
        b = wrefs[2 * i + 1]
        x = _mm(x, w[:, :]) + b[0:1, :]
        x = jnp.maximum(x, 0.0) if acts[i] == 'r' else jnp.tanh(x)
    out_ref[0, :, :] = x


def _fp_stage(xyz1, xyz2, points1, points2, layers, acts, tile):
    bb, n1, _ = xyz1.shape
    n2 = xyz2.shape[1]
    c1 = points1.shape[2]
    c2 = points2.shape[2]
    out_dim = layers[-1][0].shape[1]
    kt = jnp.swapaxes(xyz2, 1, 2)

    args = [xyz1, kt, points1, points2]
    in_specs = [
        pl.BlockSpec((1, tile, 3), lambda b, s: (b, s, 0)),
        pl.BlockSpec((1, 3, n2), lambda b, s: (b, 0, 0)),
        pl.BlockSpec((1, tile, c1), lambda b, s: (b, s, 0)),
        pl.BlockSpec((1, n2, c2), lambda b, s: (b, 0, 0)),
    ]
    for w, b in layers:
        args += [w, b.reshape(1, -1)]
        in_specs.append(pl.BlockSpec(w.shape, lambda b, s: (0, 0)))
        in_specs.append(pl.BlockSpec((1, w.shape[1]), lambda b, s: (0, 0)))

    return pl.pallas_call(
        functools.partial(_stage_kernel, len(layers), c1, acts),
        grid=(bb, n1 // tile),
        in_specs=in_specs,
        out_specs=pl.BlockSpec((1, tile, out_dim), lambda b, s: (b, s, 0)),
        out_shape=jax.ShapeDtypeStruct((bb, n1, out_dim), jnp.float32),
        compiler_params=pltpu.CompilerParams(
            dimension_semantics=("parallel", "arbitrary")),
    )(*args)


def kernel(xyz0, xyz1, xyz2, xyz3, xyz4, points0, points1, points2, points3,
           points4, fa1_W0, fa1_b0, fa1_W1, fa1_b1, fa2_W0, fa2_b0, fa2_W1,
           fa2_b1, fa3_W0, fa3_b0, fa3_W1, fa3_b1, fa4_W0, fa4_b0, fa4_W1,
           fa4_b1, fa4_W2, fa4_b2, head_W0, head_b0, head_W1, head_b1,
           head_W2, head_b2):
    p3 = _fp_stage(xyz3, xyz4, points3, points4,
                   [(fa1_W0, fa1_b0), (fa1_W1, fa1_b1)], 'rr', 64)
    p2 = _fp_stage(xyz2, xyz3, points2, p3,
                   [(fa2_W0, fa2_b0), (fa2_W1, fa2_b1)], 'rr', 256)
    p1 = _fp_stage(xyz1, xyz2, points1, p2,
                   [(fa3_W0, fa3_b0), (fa3_W1, fa3_b1)], 'rr', 1024)
    head_W2p = jnp.pad(head_W2, ((0, 0), (0, 5)))
    head_b2p = jnp.pad(head_b2, (0, 5))
    out8 = _fp_stage(xyz0, xyz1, points0, p1,
                     [(fa4_W0, fa4_b0), (fa4_W1, fa4_b1), (fa4_W2, fa4_b2),
                      (head_W0, head_b0), (head_W1, head_b1),
                      (head_W2p, head_b2p)], 'rrrrrt', 512)
    return out8[..., :3]


# speedup vs baseline: 1.3245x; 1.0034x over previous
import functools

import jax
import jax.numpy as jnp
from jax.experimental import pallas as pl
from jax.experimental.pallas import tpu as pltpu


def _mm(x, w):
    return jnp.dot(x.astype(jnp.bfloat16), w.astype(jnp.bfloat16),
                   preferred_element_type=jnp.float32)


def _limbs(x):
    a = x.astype(jnp.bfloat16)
    r = x - a.astype(jnp.float32)
    b = r.astype(jnp.bfloat16)
    c = (r - b.astype(jnp.float32)).astype(jnp.bfloat16)
    return a, b, c


def _stage_kernel(nlayers, c1, acts, emit_limbs, *refs):
    q_ref, kt_ref, p1_ref, p2cat_ref = refs[:4]
    wrefs = refs[4:4 + 2 * nlayers]
    out_ref = refs[-1]

    t = q_ref.shape[1]
    n2 = kt_ref.shape[2]
    f32 = jnp.float32

    q = q_ref[0, :, :]
    kt = kt_ref[0, :, :]
    aa = jnp.sum(q * q, axis=1, keepdims=True)
    bb = jnp.sum(kt * kt, axis=0, keepdims=True)
    ab = _mm(q, kt)
    d = jnp.maximum(aa + bb - 2.0 * ab, 0.0)

    iota = jax.lax.broadcasted_iota(jnp.int32, (t, n2), 1).astype(f32)
    big = jnp.float32(3.0e38)
    nf = jnp.float32(n2)
    dw = d
    ohs = []
    wks = []
    for k in range(3):
        mn = jnp.min(dw, axis=1, keepdims=True)
        am = jnp.min(jnp.where(dw == mn, iota, nf), axis=1, keepdims=True)
        oh = iota == am
        wks.append(1.0 / jnp.maximum(mn, 1e-10))
        ohs.append(oh.astype(jnp.bfloat16))
        if k < 2:
            dw = jnp.where(oh, big, dw)
    wtot = wks[0] + wks[1] + wks[2]

    p2cat = p2cat_ref[0, :, :]
    c2 = p2cat.shape[1] // 3
    ohcat = jnp.concatenate(ohs, axis=0)
    gcat = jnp.dot(ohcat, p2cat, preferred_element_type=f32)
    interp = None
    for k in range(3):
        gk = gcat[k * t:(k + 1) * t, :]
        g = gk[:, :c2] + gk[:, c2:2 * c2] + gk[:, 2 * c2:]
        term = g * (wks[k] / wtot)
        interp = term if interp is None else interp + term

    w0 = wrefs[0]
    b0 = wrefs[1]
    x = (_mm(p1_ref[0, :, :], w0[:c1, :]) + _mm(interp, w0[c1:, :])
         + b0[0:1, :])
    x = jnp.maximum(x, 0.0) if acts[0] == 'r' else jnp.tanh(x)
    for i in range(1, nlayers):
        w = wrefs[2 * i]
        b = wrefs[2 * i + 1]
        x = _mm(x, w[:, :]) + b[0:1, :]
        x = jnp.maximum(x, 0.0) if acts[i] == 'r' else jnp.tanh(x)
    if emit_limbs:
        xa, xb, xc = _limbs(x)
        out_ref[0, :, :] = jnp.concatenate([xa, xb, xc], axis=1)
    else:
        out_ref[0, :, :] = x


def _fp_stage(xyz1, xyz2, points1, p2cat, layers, acts, tile,
              emit_limbs=False):
    bb_, n1, _ = xyz1.shape
    n2 = xyz2.shape[1]
    c1 = points1.shape[2]
    c2cat = p2cat.shape[2]
    out_dim = layers[-1][0].shape[1]
    kt = jnp.swapaxes(xyz2, 1, 2)

    args = [xyz1, kt, points1, p2cat]
    in_specs = [
        pl.BlockSpec((1, tile, 3), lambda b, s: (b, s, 0)),
        pl.BlockSpec((1, 3, n2), lambda b, s: (b, 0, 0)),
        pl.BlockSpec((1, tile, c1), lambda b, s: (b, s, 0)),
        pl.BlockSpec((1, n2, c2cat), lambda b, s: (b, 0, 0)),
    ]
    for w, b in layers:
        args += [w, b.reshape(1, -1)]
        in_specs.append(pl.BlockSpec(w.shape, lambda b, s: (0, 0)))
        in_specs.append(pl.BlockSpec((1, w.shape[1]), lambda b, s: (0, 0)))

    if emit_limbs:
        out_shape = jax.ShapeDtypeStruct((bb_, n1, 3 * out_dim), jnp.bfloat16)
        out_spec = pl.BlockSpec((1, tile, 3 * out_dim), lambda b, s: (b, s, 0))
    else:
        out_shape = jax.ShapeDtypeStruct((bb_, n1, out_dim), jnp.float32)
        out_spec = pl.BlockSpec((1, tile, out_dim), lambda b, s: (b, s, 0))

    return pl.pallas_call(
        functools.partial(_stage_kernel, len(layers), c1, acts, emit_limbs),
        grid=(bb_, n1 // tile),
        in_specs=in_specs,
        out_specs=out_spec,
        out_shape=out_shape,
        compiler_params=pltpu.CompilerParams(
            dimension_semantics=("parallel", "arbitrary")),
    )(*args)


def kernel(xyz0, xyz1, xyz2, xyz3, xyz4, points0, points1, points2, points3,
           points4, fa1_W0, fa1_b0, fa1_W1, fa1_b1, fa2_W0, fa2_b0, fa2_W1,
           fa2_b1, fa3_W0, fa3_b0, fa3_W1, fa3_b1, fa4_W0, fa4_b0, fa4_W1,
           fa4_b1, fa4_W2, fa4_b2, head_W0, head_b0, head_W1, head_b1,
           head_W2, head_b2):
    p4a = points4.astype(jnp.bfloat16)
    r = points4 - p4a.astype(jnp.float32)
    p4b = r.astype(jnp.bfloat16)
    p4c = (r - p4b.astype(jnp.float32)).astype(jnp.bfloat16)
    p4cat = jnp.concatenate([p4a, p4b, p4c], axis=2)

    p3cat = _fp_stage(xyz3, xyz4, points3, p4cat,
                      [(fa1_W0, fa1_b0), (fa1_W1, fa1_b1)], 'rr', 64,
                      emit_limbs=True)
    p2cat = _fp_stage(xyz2, xyz3, points2, p3cat,
                      [(fa2_W0, fa2_b0), (fa2_W1, fa2_b1)], 'rr', 256,
                      emit_limbs=True)
    p1cat = _fp_stage(xyz1, xyz2, points1, p2cat,
                      [(fa3_W0, fa3_b0), (fa3_W1, fa3_b1)], 'rr', 1024,
                      emit_limbs=True)
    head_W2p = jnp.pad(head_W2, ((0, 0), (0, 5)))
    head_b2p = jnp.pad(head_b2, (0, 5))
    out8 = _fp_stage(xyz0, xyz1, points0, p1cat,
                     [(fa4_W0, fa4_b0), (fa4_W1, fa4_b1), (fa4_W2, fa4_b2),
                      (head_W0, head_b0), (head_W1, head_b1),
                      (head_W2p, head_b2p)], 'rrrrrt', 512)
    return out8[..., :3]


# speedup vs baseline: 1.4902x; 1.1251x over previous
import functools

import jax
import jax.numpy as jnp
from jax.experimental import pallas as pl
from jax.experimental.pallas import tpu as pltpu


def _mm(x, w):
    return jnp.dot(x.astype(jnp.bfloat16), w.astype(jnp.bfloat16),
                   preferred_element_type=jnp.float32)


def _limbs(x):
    a = x.astype(jnp.bfloat16)
    r = x - a.astype(jnp.float32)
    b = r.astype(jnp.bfloat16)
    c = (r - b.astype(jnp.float32)).astype(jnp.bfloat16)
    return a, b, c


def _stage_kernel(nlayers, c1, acts, *refs):
    q_ref, kt_ref, p1_ref, p2cat_ref = refs[:4]
    wrefs = refs[4:4 + 2 * nlayers]
    out_ref = refs[-1]

    t = q_ref.shape[1]
    n2 = kt_ref.shape[2]
    f32 = jnp.float32

    q = q_ref[0, :, :]
    kt = kt_ref[0, :, :]
    aa = jnp.sum(q * q, axis=1, keepdims=True)
    bb = jnp.sum(kt * kt, axis=0, keepdims=True)
    ab = _mm(q, kt)
    d = jnp.maximum(aa + bb - 2.0 * ab, 0.0)

    iota = jax.lax.broadcasted_iota(jnp.int32, (t, n2), 1).astype(f32)
    big = jnp.float32(3.0e38)
    nf = jnp.float32(n2)
    dw = d
    ohs = []
    wks = []
    for k in range(3):
        mn = jnp.min(dw, axis=1, keepdims=True)
        am = jnp.min(jnp.where(dw == mn, iota, nf), axis=1, keepdims=True)
        oh = iota == am
        wks.append(1.0 / jnp.maximum(mn, 1e-10))
        ohs.append(oh.astype(jnp.bfloat16))
        if k < 2:
            dw = jnp.where(oh, big, dw)
    wtot = wks[0] + wks[1] + wks[2]

    p2 = p2cat_ref[0, :, :]
    p2a, p2b, p2c = _limbs(p2)
    p2cat = jnp.concatenate([p2a, p2b, p2c], axis=1)
    c2 = p2.shape[1]
    ohcat = jnp.concatenate(ohs, axis=0)
    gcat = jnp.dot(ohcat, p2cat, preferred_element_type=f32)
    interp = None
    for k in range(3):
        gk = gcat[k * t:(k + 1) * t, :]
        g = gk[:, :c2] + gk[:, c2:2 * c2] + gk[:, 2 * c2:]
        term = g * (wks[k] / wtot)
        interp = term if interp is None else interp + term

    w0 = wrefs[0]
    b0 = wrefs[1]
    x = (_mm(p1_ref[0, :, :], w0[:c1, :]) + _mm(interp, w0[c1:, :])
         + b0[0:1, :])
    x = jnp.maximum(x, 0.0) if acts[0] == 'r' else jnp.tanh(x)
    for i in range(1, nlayers):
        w = wrefs[2 * i]
        b = wrefs[2 * i + 1]
        x = _mm(x, w[:, :]) + b[0:1, :]
        x = jnp.maximum(x, 0.0) if acts[i] == 'r' else jnp.tanh(x)
    out_ref[0, :, :] = x


def _fp_stage(xyz1, xyz2, points1, points2, layers, acts, tile):
    bb_, n1, _ = xyz1.shape
    n2 = xyz2.shape[1]
    c1 = points1.shape[2]
    c2in = points2.shape[2]
    out_dim = layers[-1][0].shape[1]
    kt = jnp.swapaxes(xyz2, 1, 2)

    args = [xyz1, kt, points1, points2]
    in_specs = [
        pl.BlockSpec((1, tile, 3), lambda b, s: (b, s, 0)),
        pl.BlockSpec((1, 3, n2), lambda b, s: (b, 0, 0)),
        pl.BlockSpec((1, tile, c1), lambda b, s: (b, s, 0)),
        pl.BlockSpec((1, n2, c2in), lambda b, s: (b, 0, 0)),
    ]
    for w, b in layers:
        args += [w, b.reshape(1, -1)]
        in_specs.append(pl.BlockSpec(w.shape, lambda b, s: (0, 0)))
        in_specs.append(pl.BlockSpec((1, w.shape[1]), lambda b, s: (0, 0)))

    out_shape = jax.ShapeDtypeStruct((bb_, n1, out_dim), jnp.float32)
    out_spec = pl.BlockSpec((1, tile, out_dim), lambda b, s: (b, s, 0))

    return pl.pallas_call(
        functools.partial(_stage_kernel, len(layers), c1, acts),
        grid=(bb_, n1 // tile),
        in_specs=in_specs,
        out_specs=out_spec,
        out_shape=out_shape,
        compiler_params=pltpu.CompilerParams(
            dimension_semantics=("parallel", "arbitrary")),
    )(*args)


def kernel(xyz0, xyz1, xyz2, xyz3, xyz4, points0, points1, points2, points3,
           points4, fa1_W0, fa1_b0, fa1_W1, fa1_b1, fa2_W0, fa2_b0, fa2_W1,
           fa2_b1, fa3_W0, fa3_b0, fa3_W1, fa3_b1, fa4_W0, fa4_b0, fa4_W1,
           fa4_b1, fa4_W2, fa4_b2, head_W0, head_b0, head_W1, head_b1,
           head_W2, head_b2):
    p3 = _fp_stage(xyz3, xyz4, points3, points4,
                   [(fa1_W0, fa1_b0), (fa1_W1, fa1_b1)], 'rr', 64)
    p2 = _fp_stage(xyz2, xyz3, points2, p3,
                   [(fa2_W0, fa2_b0), (fa2_W1, fa2_b1)], 'rr', 256)
    p1 = _fp_stage(xyz1, xyz2, points1, p2,
                   [(fa3_W0, fa3_b0), (fa3_W1, fa3_b1)], 'rr', 1024)
    head_W2p = jnp.pad(head_W2, ((0, 0), (0, 5)))
    head_b2p = jnp.pad(head_b2, (0, 5))
    out8 = _fp_stage(xyz0, xyz1, points0, p1,
                     [(fa4_W0, fa4_b0), (fa4_W1, fa4_b1), (fa4_W2, fa4_b2),
                      (head_W0, head_b0), (head_W1, head_b1),
                      (head_W2p, head_b2p)], 'rrrrrt', 1024)
    return out8[..., :3]


# speedup vs baseline: 1.5486x; 1.0392x over previous
import functools

import jax
import jax.numpy as jnp
from jax.experimental import pallas as pl
from jax.experimental.pallas import tpu as pltpu


def _mm(x, w):
    return jnp.dot(x.astype(jnp.bfloat16), w.astype(jnp.bfloat16),
                   preferred_element_type=jnp.float32)


def _limbs(x):
    a = x.astype(jnp.bfloat16)
    r = x - a.astype(jnp.float32)
    b = r.astype(jnp.bfloat16)
    c = (r - b.astype(jnp.float32)).astype(jnp.bfloat16)
    return a, b, c


def _stage_kernel(nlayers, c1, acts, *refs):
    q_ref, kt_ref, p1_ref, p2cat_ref = refs[:4]
    wrefs = refs[4:4 + 2 * nlayers]
    out_ref = refs[-1]

    t = q_ref.shape[1]
    n2 = kt_ref.shape[2]
    f32 = jnp.float32

    q = q_ref[0, :, :]
    kt = kt_ref[0, :, :]
    aa = jnp.sum(q * q, axis=1, keepdims=True)
    bb = jnp.sum(kt * kt, axis=0, keepdims=True)
    ab = _mm(q, kt)
    d = jnp.maximum(aa + bb - 2.0 * ab, 0.0)

    iota = jax.lax.broadcasted_iota(jnp.int32, (t, n2), 1).astype(f32)
    big = jnp.float32(3.0e38)
    nf = jnp.float32(n2)
    dw = d
    ohs = []
    wks = []
    for k in range(3):
        mn = jnp.min(dw, axis=1, keepdims=True)
        am = jnp.min(jnp.where(dw == mn, iota, nf), axis=1, keepdims=True)
        oh = iota == am
        wks.append(1.0 / jnp.maximum(mn, 1e-10))
        ohs.append(oh.astype(jnp.bfloat16))
        if k < 2:
            dw = jnp.where(oh, big, dw)
    wtot = wks[0] + wks[1] + wks[2]

    p2 = p2cat_ref[0, :, :]
    p2a, p2b, p2c = _limbs(p2)
    p2cat = jnp.concatenate([p2a, p2b, p2c], axis=1)
    c2 = p2.shape[1]
    ohcat = jnp.concatenate(ohs, axis=0)
    gcat = jnp.dot(ohcat, p2cat, preferred_element_type=f32)
    interp = None
    for k in range(3):
        gk = gcat[k * t:(k + 1) * t, :]
        g = gk[:, :c2] + gk[:, c2:2 * c2] + gk[:, 2 * c2:]
        term = g * (wks[k] / wtot)
        interp = term if interp is None else interp + term

    w0 = wrefs[0]
    b0 = wrefs[1]
    x = (_mm(p1_ref[0, :, :], w0[:c1, :]) + _mm(interp, w0[c1:, :])
         + b0[0:1, :])
    x = jnp.maximum(x, 0.0) if acts[0] == 'r' else jnp.tanh(x)
    for i in range(1, nlayers):
        w = wrefs[2 * i]
        b = wrefs[2 * i + 1]
        x = _mm(x, w[:, :]) + b[0:1, :]
        x = jnp.maximum(x, 0.0) if acts[i] == 'r' else jnp.tanh(x)
    out_ref[0, :, :] = x


def _fp_stage(xyz1, xyz2, points1, points2, layers, acts, tile):
    bb_, n1, _ = xyz1.shape
    n2 = xyz2.shape[1]
    c1 = points1.shape[2]
    c2in = points2.shape[2]
    out_dim = layers[-1][0].shape[1]
    kt = jnp.swapaxes(xyz2, 1, 2)

    args = [xyz1, kt, points1, points2]
    in_specs = [
        pl.BlockSpec((1, tile, 3), lambda b, s: (b, s, 0)),
        pl.BlockSpec((1, 3, n2), lambda b, s: (b, 0, 0)),
        pl.BlockSpec((1, tile, c1), lambda b, s: (b, s, 0)),
        pl.BlockSpec((1, n2, c2in), lambda b, s: (b, 0, 0)),
    ]
    for w, b in layers:
        args += [w, b.reshape(1, -1)]
        in_specs.append(pl.BlockSpec(w.shape, lambda b, s: (0, 0)))
        in_specs.append(pl.BlockSpec((1, w.shape[1]), lambda b, s: (0, 0)))

    out_shape = jax.ShapeDtypeStruct((bb_, n1, out_dim), jnp.float32)
    out_spec = pl.BlockSpec((1, tile, out_dim), lambda b, s: (b, s, 0))

    return pl.pallas_call(
        functools.partial(_stage_kernel, len(layers), c1, acts),
        grid=(bb_, n1 // tile),
        in_specs=in_specs,
        out_specs=out_spec,
        out_shape=out_shape,
        compiler_params=pltpu.CompilerParams(
            dimension_semantics=("parallel", "arbitrary")),
    )(*args)


def kernel(xyz0, xyz1, xyz2, xyz3, xyz4, points0, points1, points2, points3,
           points4, fa1_W0, fa1_b0, fa1_W1, fa1_b1, fa2_W0, fa2_b0, fa2_W1,
           fa2_b1, fa3_W0, fa3_b0, fa3_W1, fa3_b1, fa4_W0, fa4_b0, fa4_W1,
           fa4_b1, fa4_W2, fa4_b2, head_W0, head_b0, head_W1, head_b1,
           head_W2, head_b2):
    p3 = _fp_stage(xyz3, xyz4, points3, points4,
                   [(fa1_W0, fa1_b0), (fa1_W1, fa1_b1)], 'rr', 64)
    p2 = _fp_stage(xyz2, xyz3, points2, p3,
                   [(fa2_W0, fa2_b0), (fa2_W1, fa2_b1)], 'rr', 256)
    p1 = _fp_stage(xyz1, xyz2, points1, p2,
                   [(fa3_W0, fa3_b0), (fa3_W1, fa3_b1)], 'rr', 1024)
    head_W2p = jnp.pad(head_W2, ((0, 0), (0, 5)))
    head_b2p = jnp.pad(head_b2, (0, 5))
    out8 = _fp_stage(xyz0, xyz1, points0, p1,
                     [(fa4_W0, fa4_b0), (fa4_W1, fa4_b1), (fa4_W2, fa4_b2),
                      (head_W0, head_b0), (head_W1, head_b1),
                      (head_W2p, head_b2p)], 'rrrrrt', 2048)
    return out8[..., :3]


# speedup vs baseline: 1.5778x; 1.0188x over previous
import functools

import jax
import jax.numpy as jnp
from jax.experimental import pallas as pl
from jax.experimental.pallas import tpu as pltpu


def _mm(x, w):
    return jnp.dot(x.astype(jnp.bfloat16), w.astype(jnp.bfloat16),
                   preferred_element_type=jnp.float32)


def _limbs(x):
    a = x.astype(jnp.bfloat16)
    r = x - a.astype(jnp.float32)
    b = r.astype(jnp.bfloat16)
    c = (r - b.astype(jnp.float32)).astype(jnp.bfloat16)
    return a, b, c


def _stage_kernel(nlayers, c1, acts, *refs):
    q_ref, kt_ref, p1_ref, p2cat_ref = refs[:4]
    wrefs = refs[4:4 + 2 * nlayers]
    out_ref = refs[-1]

    t = q_ref.shape[1]
    n2 = kt_ref.shape[2]
    f32 = jnp.float32

    q = q_ref[0, :, :]
    kt = kt_ref[0, :, :]
    aa = jnp.sum(q * q, axis=1, keepdims=True)
    bb = jnp.sum(kt * kt, axis=0, keepdims=True)
    ab = _mm(q, kt)
    d = jnp.maximum(aa + bb - 2.0 * ab, 0.0)

    iota = jax.lax.broadcasted_iota(jnp.int32, (t, n2), 1).astype(f32)
    big = jnp.float32(3.0e38)
    nf = jnp.float32(n2)
    dw = d
    ohs = []
    wks = []
    for k in range(3):
        mn = jnp.min(dw, axis=1, keepdims=True)
        am = jnp.min(jnp.where(dw == mn, iota, nf), axis=1, keepdims=True)
        oh = iota == am
        wks.append(1.0 / jnp.maximum(mn, 1e-10))
        ohs.append(oh.astype(jnp.bfloat16))
        if k < 2:
            dw = jnp.where(oh, big, dw)
    wtot = wks[0] + wks[1] + wks[2]

    p2 = p2cat_ref[0, :, :]
    p2a, p2b, p2c = _limbs(p2)
    p2cat = jnp.concatenate([p2a, p2b, p2c], axis=1)
    c2 = p2.shape[1]
    ohcat = jnp.concatenate(ohs, axis=0)
    gcat = jnp.dot(ohcat, p2cat, preferred_element_type=f32)
    interp = None
    for k in range(3):
        gk = gcat[k * t:(k + 1) * t, :]
        g = gk[:, :c2] + gk[:, c2:2 * c2] + gk[:, 2 * c2:]
        term = g * (wks[k] / wtot)
        interp = term if interp is None else interp + term

    w0 = wrefs[0]
    b0 = wrefs[1]
    x = (_mm(p1_ref[0, :, :], w0[:c1, :]) + _mm(interp, w0[c1:, :])
         + b0[0:1, :])
    x = jnp.maximum(x, 0.0) if acts[0] == 'r' else jnp.tanh(x)
    for i in range(1, nlayers):
        w = wrefs[2 * i]
        b = wrefs[2 * i + 1]
        x = _mm(x, w[:, :]) + b[0:1, :]
        x = jnp.maximum(x, 0.0) if acts[i] == 'r' else jnp.tanh(x)
    out_ref[0, :, :] = x


def _fp_stage(xyz1, xyz2, points1, points2, layers, acts, tile):
    bb_, n1, _ = xyz1.shape
    n2 = xyz2.shape[1]
    c1 = points1.shape[2]
    c2in = points2.shape[2]
    out_dim = layers[-1][0].shape[1]
    kt = jnp.swapaxes(xyz2, 1, 2)

    args = [xyz1, kt, points1, points2]
    in_specs = [
        pl.BlockSpec((1, tile, 3), lambda b, s: (b, s, 0)),
        pl.BlockSpec((1, 3, n2), lambda b, s: (b, 0, 0)),
        pl.BlockSpec((1, tile, c1), lambda b, s: (b, s, 0)),
        pl.BlockSpec((1, n2, c2in), lambda b, s: (b, 0, 0)),
    ]
    for w, b in layers:
        args += [w, b.reshape(1, -1)]
        in_specs.append(pl.BlockSpec(w.shape, lambda b, s: (0, 0)))
        in_specs.append(pl.BlockSpec((1, w.shape[1]), lambda b, s: (0, 0)))

    out_shape = jax.ShapeDtypeStruct((bb_, n1, out_dim), jnp.float32)
    out_spec = pl.BlockSpec((1, tile, out_dim), lambda b, s: (b, s, 0))

    return pl.pallas_call(
        functools.partial(_stage_kernel, len(layers), c1, acts),
        grid=(bb_, n1 // tile),
        in_specs=in_specs,
        out_specs=out_spec,
        out_shape=out_shape,
        compiler_params=pltpu.CompilerParams(
            dimension_semantics=("parallel", "arbitrary")),
    )(*args)


def kernel(xyz0, xyz1, xyz2, xyz3, xyz4, points0, points1, points2, points3,
           points4, fa1_W0, fa1_b0, fa1_W1, fa1_b1, fa2_W0, fa2_b0, fa2_W1,
           fa2_b1, fa3_W0, fa3_b0, fa3_W1, fa3_b1, fa4_W0, fa4_b0, fa4_W1,
           fa4_b1, fa4_W2, fa4_b2, head_W0, head_b0, head_W1, head_b1,
           head_W2, head_b2):
    p3 = _fp_stage(xyz3, xyz4, points3, points4,
                   [(fa1_W0, fa1_b0), (fa1_W1, fa1_b1)], 'rr', 64)
    p2 = _fp_stage(xyz2, xyz3, points2, p3,
                   [(fa2_W0, fa2_b0), (fa2_W1, fa2_b1)], 'rr', 256)
    p1 = _fp_stage(xyz1, xyz2, points1, p2,
                   [(fa3_W0, fa3_b0), (fa3_W1, fa3_b1)], 'rr', 1024)
    head_W2p = jnp.pad(head_W2, ((0, 0), (0, 5)))
    head_b2p = jnp.pad(head_b2, (0, 5))
    out8 = _fp_stage(xyz0, xyz1, points0, p1,
                     [(fa4_W0, fa4_b0), (fa4_W1, fa4_b1), (fa4_W2, fa4_b2),
                      (head_W0, head_b0), (head_W1, head_b1),
                      (head_W2p, head_b2p)], 'rrrrrt', 4096)
    return out8[..., :3]
